# baseline (speedup 1.0000x reference)
import functools

import jax
import jax.numpy as jnp
from jax.experimental import pallas as pl
from jax.experimental.pallas import tpu as pltpu

_ROWS = 512


def _coords_and_weights(grid_blk, D, H, W):
    ix = (grid_blk[:, 0:1] + 1.0) * (0.5 * (W - 1))
    iy = (grid_blk[:, 1:2] + 1.0) * (0.5 * (H - 1))
    iz = (grid_blk[:, 2:3] + 1.0) * (0.5 * (D - 1))
    ix0f = jnp.floor(ix)
    iy0f = jnp.floor(iy)
    iz0f = jnp.floor(iz)
    tx = ix - ix0f
    ty = iy - iy0f
    tz = iz - iz0f
    return (ix0f.astype(jnp.int32), tx), (iy0f.astype(jnp.int32), ty), (iz0f.astype(jnp.int32), tz)


def _tap_weight(idx_lane, i0, t):
    return (jnp.where(idx_lane == i0, 1.0 - t, 0.0)
            + jnp.where(idx_lane == i0 + 1, t, 0.0))


def _tap_dweight(idx_lane, i0):
    return (jnp.where(idx_lane == i0 + 1, 1.0, 0.0)
            - jnp.where(idx_lane == i0, 1.0, 0.0))


def _zy_indices(PB, row0, H):
    zy = jax.lax.broadcasted_iota(jnp.int32, (PB, _ROWS), 1) + row0
    return zy // H, zy % H


def _replicate_channels(go_blk, C, W):
    lane = jax.lax.broadcasted_iota(jnp.int32, (C, W * C), 1)
    sub = jax.lax.broadcasted_iota(jnp.int32, (C, W * C), 0)
    B = ((lane % C) == sub).astype(jnp.float32)
    return jax.lax.dot(go_blk, B, precision=jax.lax.Precision.HIGHEST,
                       preferred_element_type=jnp.float32)


def _grad_input_kernel(grid_ref, go_ref, gi_ref, *, D, H, W, C, PB):
    zc = pl.program_id(1)
    pb = pl.program_id(2)

    (ix0, tx), (iy0, ty), (iz0, tz) = _coords_and_weights(grid_ref[...], D, H, W)

    z_i, y_i = _zy_indices(PB, zc * _ROWS, H)
    mzy = _tap_weight(z_i, iz0, tz) * _tap_weight(y_i, iy0, ty)

    xc = jax.lax.broadcasted_iota(jnp.int32, (PB, W * C), 1)
    wx = _tap_weight(xc // C, ix0, tx)
    k = wx * _replicate_channels(go_ref[...], C, W)

    @pl.when(pb == 0)
    def _():
        gi_ref[...] = jnp.zeros_like(gi_ref)

    gi_ref[...] += jax.lax.dot_general(
        mzy, k, (((0,), (0,)), ((), ())),
        precision=jax.lax.Precision.HIGHEST,
        preferred_element_type=jnp.float32)


def _grad_grid_kernel(grid_ref, go_ref, inp_ref, gx_ref, gy_ref, gz_ref,
                      u0_ref, u1_ref, u2_ref, *, D, H, W, C, PB, KC):
    kc = pl.program_id(2)

    (ix0, tx), (iy0, ty), (iz0, tz) = _coords_and_weights(grid_ref[...], D, H, W)

    z_i, y_i = _zy_indices(PB, kc * _ROWS, H)
    wz = _tap_weight(z_i, iz0, tz)
    wy = _tap_weight(y_i, iy0, ty)
    dwz = _tap_dweight(z_i, iz0)
    dwy = _tap_dweight(y_i, iy0)

    a = inp_ref[...]

    def contract(m):
        return jax.lax.dot(m, a, precision=jax.lax.Precision.HIGHEST,
                           preferred_element_type=jnp.float32)

    @pl.when(kc == 0)
    def _():
        u0_ref[...] = jnp.zeros_like(u0_ref)
        u1_ref[...] = jnp.zeros_like(u1_ref)
        u2_ref[...] = jnp.zeros_like(u2_ref)

    u0_ref[...] += contract(wz * wy)
    u1_ref[...] += contract(wz * dwy)
    u2_ref[...] += contract(dwz * wy)

    @pl.when(kc == KC - 1)
    def _():
        xc = jax.lax.broadcasted_iota(jnp.int32, (PB, W * C), 1)
        go_rep = _replicate_channels(go_ref[...], C, W)
        k = _tap_weight(xc // C, ix0, tx) * go_rep
        dk = _tap_dweight(xc // C, ix0) * go_rep
        gx_ref[...] = jnp.sum(u0_ref[...] * dk, axis=1, keepdims=True) * (0.5 * (W - 1))
        gy_ref[...] = jnp.sum(u1_ref[...] * k, axis=1, keepdims=True) * (0.5 * (H - 1))
        gz_ref[...] = jnp.sum(u2_ref[...] * k, axis=1, keepdims=True) * (0.5 * (D - 1))


@jax.jit
def _run(grad_output, input, grid):
    N, C, D, H, W = input.shape
    _, Do, Ho, Wo, _ = grid.shape
    P = Do * Ho * Wo

    grid_f = grid.reshape(N, P, 3)
    go_f = jnp.transpose(grad_output.reshape(N, C, P), (0, 2, 1))
    inp_f = jnp.transpose(input, (0, 2, 3, 4, 1)).reshape(N, D * H, W * C)

    n_chunks = (D * H) // _ROWS

    PB1 = 512
    gi_flat = pl.pallas_call(
        functools.partial(_grad_input_kernel, D=D, H=H, W=W, C=C, PB=PB1),
        grid=(N, n_chunks, P // PB1),
        in_specs=[
            pl.BlockSpec((None, PB1, 3), lambda n, zc, p: (n, p, 0)),
            pl.BlockSpec((None, PB1, C), lambda n, zc, p: (n, p, 0)),
        ],
        out_specs=pl.BlockSpec((None, _ROWS, W * C), lambda n, zc, p: (n, zc, 0)),
        out_shape=jax.ShapeDtypeStruct((N, D * H, W * C), jnp.float32),
        compiler_params=pltpu.CompilerParams(
            dimension_semantics=("parallel", "arbitrary", "arbitrary")),
    )(grid_f, go_f)
    grad_input = jnp.transpose(
        gi_flat.reshape(N, D, H, W, C), (0, 4, 1, 2, 3))

    PB2 = 512
    out_sds = jax.ShapeDtypeStruct((N, P, 1), jnp.float32)
    out_spec = pl.BlockSpec((None, PB2, 1), lambda n, p, kc: (n, p, 0))
    u_scratch = pltpu.VMEM((PB2, W * C), jnp.float32)
    gx, gy, gz = pl.pallas_call(
        functools.partial(_grad_grid_kernel, D=D, H=H, W=W, C=C, PB=PB2,
                          KC=n_chunks),
        grid=(N, P // PB2, n_chunks),
        in_specs=[
            pl.BlockSpec((None, PB2, 3), lambda n, p, kc: (n, p, 0)),
            pl.BlockSpec((None, PB2, C), lambda n, p, kc: (n, p, 0)),
            pl.BlockSpec((None, _ROWS, W * C), lambda n, p, kc: (n, kc, 0)),
        ],
        out_specs=(out_spec, out_spec, out_spec),
        out_shape=(out_sds, out_sds, out_sds),
        scratch_shapes=[u_scratch, u_scratch, u_scratch],
        compiler_params=pltpu.CompilerParams(
            dimension_semantics=("parallel", "arbitrary", "arbitrary")),
    )(grid_f, go_f, inp_f)
    grad_grid = jnp.concatenate([gx, gy, gz], axis=-1).reshape(N, Do, Ho, Wo, 3)

    return grad_input, grad_grid


def kernel(grad_output, input, grid, interpolation_mode, padding_mode,
           align_corners, output_mask):
    return _run(grad_output, input, grid)


# speedup vs baseline: 3.7642x; 3.7642x over previous
import functools

import jax
import jax.numpy as jnp
from jax.experimental import pallas as pl
from jax.experimental.pallas import tpu as pltpu

_ROWS = 512


def _coords_and_weights(grid_blk, D, H, W):
    ix = (grid_blk[:, 0:1] + 1.0) * (0.5 * (W - 1))
    iy = (grid_blk[:, 1:2] + 1.0) * (0.5 * (H - 1))
    iz = (grid_blk[:, 2:3] + 1.0) * (0.5 * (D - 1))
    ix0f = jnp.floor(ix)
    iy0f = jnp.floor(iy)
    iz0f = jnp.floor(iz)
    tx = ix - ix0f
    ty = iy - iy0f
    tz = iz - iz0f
    return (ix0f.astype(jnp.int32), tx), (iy0f.astype(jnp.int32), ty), (iz0f.astype(jnp.int32), tz)


def _tap_weight(idx_lane, i0, t):
    return (jnp.where(idx_lane == i0, 1.0 - t, 0.0)
            + jnp.where(idx_lane == i0 + 1, t, 0.0))


def _tap_dweight(idx_lane, i0):
    return (jnp.where(idx_lane == i0 + 1, 1.0, 0.0)
            - jnp.where(idx_lane == i0, 1.0, 0.0))


def _zy_indices(PB, row0, H):
    zy = jax.lax.broadcasted_iota(jnp.int32, (PB, _ROWS), 1) + row0
    return zy // H, zy % H


def _replicate_channels(go_blk, C, W):
    lane = jax.lax.broadcasted_iota(jnp.int32, (C, W * C), 1)
    sub = jax.lax.broadcasted_iota(jnp.int32, (C, W * C), 0)
    B = ((lane % C) == sub).astype(jnp.float32)
    return jax.lax.dot(go_blk, B, preferred_element_type=jnp.float32)


def _grad_input_kernel(grid_ref, go_ref, gi_ref, *, D, H, W, C, PB):
    zc = pl.program_id(1)
    pb = pl.program_id(2)

    (ix0, tx), (iy0, ty), (iz0, tz) = _coords_and_weights(grid_ref[...], D, H, W)

    z_i, y_i = _zy_indices(PB, zc * _ROWS, H)
    mzy = _tap_weight(z_i, iz0, tz) * _tap_weight(y_i, iy0, ty)

    xc = jax.lax.broadcasted_iota(jnp.int32, (PB, W * C), 1)
    wx = _tap_weight(xc // C, ix0, tx)
    k = wx * _replicate_channels(go_ref[...], C, W)

    @pl.when(pb == 0)
    def _():
        gi_ref[...] = jnp.zeros_like(gi_ref)

    gi_ref[...] += jax.lax.dot_general(
        mzy, k, (((0,), (0,)), ((), ())),
        preferred_element_type=jnp.float32)


def _grad_grid_kernel(grid_ref, go_ref, inp_ref, gx_ref, gy_ref, gz_ref,
                      u0_ref, u1_ref, u2_ref, *, D, H, W, C, PB, KC):
    kc = pl.program_id(2)

    (ix0, tx), (iy0, ty), (iz0, tz) = _coords_and_weights(grid_ref[...], D, H, W)

    z_i, y_i = _zy_indices(PB, kc * _ROWS, H)
    wz = _tap_weight(z_i, iz0, tz)
    wy = _tap_weight(y_i, iy0, ty)
    dwz = _tap_dweight(z_i, iz0)
    dwy = _tap_dweight(y_i, iy0)

    a = inp_ref[...]

    @pl.when(kc == 0)
    def _():
        u0_ref[...] = jnp.zeros_like(u0_ref)
        u1_ref[...] = jnp.zeros_like(u1_ref)
        u2_ref[...] = jnp.zeros_like(u2_ref)

    m_all = jnp.concatenate([wz * wy, wz * dwy, dwz * wy], axis=0)
    u_all = jax.lax.dot(m_all, a, preferred_element_type=jnp.float32)
    u0_ref[...] += u_all[:PB, :]
    u1_ref[...] += u_all[PB:2 * PB, :]
    u2_ref[...] += u_all[2 * PB:, :]

    @pl.when(kc == KC - 1)
    def _():
        xc = jax.lax.broadcasted_iota(jnp.int32, (PB, W * C), 1)
        go_rep = _replicate_channels(go_ref[...], C, W)
        k = _tap_weight(xc // C, ix0, tx) * go_rep
        dk = _tap_dweight(xc // C, ix0) * go_rep
        gx_ref[...] = jnp.sum(u0_ref[...] * dk, axis=1, keepdims=True) * (0.5 * (W - 1))
        gy_ref[...] = jnp.sum(u1_ref[...] * k, axis=1, keepdims=True) * (0.5 * (H - 1))
        gz_ref[...] = jnp.sum(u2_ref[...] * k, axis=1, keepdims=True) * (0.5 * (D - 1))


@jax.jit
def _run(grad_output, input, grid):
    N, C, D, H, W = input.shape
    _, Do, Ho, Wo, _ = grid.shape
    P = Do * Ho * Wo

    grid_f = grid.reshape(N, P, 3)
    go_f = jnp.transpose(grad_output.reshape(N, C, P), (0, 2, 1))
    inp_f = jnp.transpose(input, (0, 2, 3, 4, 1)).reshape(N, D * H, W * C)

    n_chunks = (D * H) // _ROWS

    PB1 = 512
    gi_flat = pl.pallas_call(
        functools.partial(_grad_input_kernel, D=D, H=H, W=W, C=C, PB=PB1),
        grid=(N, n_chunks, P // PB1),
        in_specs=[
            pl.BlockSpec((None, PB1, 3), lambda n, zc, p: (n, p, 0)),
            pl.BlockSpec((None, PB1, C), lambda n, zc, p: (n, p, 0)),
        ],
        out_specs=pl.BlockSpec((None, _ROWS, W * C), lambda n, zc, p: (n, zc, 0)),
        out_shape=jax.ShapeDtypeStruct((N, D * H, W * C), jnp.float32),
        compiler_params=pltpu.CompilerParams(
            dimension_semantics=("parallel", "arbitrary", "arbitrary")),
    )(grid_f, go_f)
    grad_input = jnp.transpose(
        gi_flat.reshape(N, D, H, W, C), (0, 4, 1, 2, 3))

    PB2 = 512
    out_sds = jax.ShapeDtypeStruct((N, P, 1), jnp.float32)
    out_spec = pl.BlockSpec((None, PB2, 1), lambda n, p, kc: (n, p, 0))
    u_scratch = pltpu.VMEM((PB2, W * C), jnp.float32)
    gx, gy, gz = pl.pallas_call(
        functools.partial(_grad_grid_kernel, D=D, H=H, W=W, C=C, PB=PB2,
                          KC=n_chunks),
        grid=(N, P // PB2, n_chunks),
        in_specs=[
            pl.BlockSpec((None, PB2, 3), lambda n, p, kc: (n, p, 0)),
            pl.BlockSpec((None, PB2, C), lambda n, p, kc: (n, p, 0)),
            pl.BlockSpec((None, _ROWS, W * C), lambda n, p, kc: (n, kc, 0)),
        ],
        out_specs=(out_spec, out_spec, out_spec),
        out_shape=(out_sds, out_sds, out_sds),
        scratch_shapes=[u_scratch, u_scratch, u_scratch],
        compiler_params=pltpu.CompilerParams(
            dimension_semantics=("parallel", "arbitrary", "arbitrary")),
    )(grid_f, go_f, inp_f)
    grad_grid = jnp.concatenate([gx, gy, gz], axis=-1).reshape(N, Do, Ho, Wo, 3)

    return grad_input, grad_grid


def kernel(grad_output, input, grid, interpolation_mode, padding_mode,
           align_corners, output_mask):
    return _run(grad_output, input, grid)


# speedup vs baseline: 4.3107x; 1.1452x over previous
import functools

import jax
import jax.numpy as jnp
from jax.experimental import pallas as pl
from jax.experimental.pallas import tpu as pltpu

_ROWS = 1024


def _coords_and_weights(grid_blk, D, H, W):
    ix = (grid_blk[:, 0:1] + 1.0) * (0.5 * (W - 1))
    iy = (grid_blk[:, 1:2] + 1.0) * (0.5 * (H - 1))
    iz = (grid_blk[:, 2:3] + 1.0) * (0.5 * (D - 1))
    ix0f = jnp.floor(ix)
    iy0f = jnp.floor(iy)
    iz0f = jnp.floor(iz)
    tx = ix - ix0f
    ty = iy - iy0f
    tz = iz - iz0f
    return (ix0f.astype(jnp.int32), tx), (iy0f.astype(jnp.int32), ty), (iz0f.astype(jnp.int32), tz)


def _tap_weight(idx_lane, i0, t):
    return (jnp.where(idx_lane == i0, 1.0 - t, 0.0)
            + jnp.where(idx_lane == i0 + 1, t, 0.0))


def _tap_dweight(idx_lane, i0):
    return (jnp.where(idx_lane == i0 + 1, 1.0, 0.0)
            - jnp.where(idx_lane == i0, 1.0, 0.0))


def _zy_indices(PB, row0, H):
    zy = jax.lax.broadcasted_iota(jnp.int32, (PB, _ROWS), 1) + row0
    return zy // H, zy % H


def _replicate_channels(go_blk, C, W):
    lane = jax.lax.broadcasted_iota(jnp.int32, (C, W * C), 1)
    sub = jax.lax.broadcasted_iota(jnp.int32, (C, W * C), 0)
    B = ((lane % C) == sub).astype(jnp.float32)
    return jax.lax.dot(go_blk, B, preferred_element_type=jnp.float32)


def _grad_input_kernel(grid_ref, go_ref, gi_ref, *, D, H, W, C, PB):
    zc = pl.program_id(1)
    pb = pl.program_id(2)

    (ix0, tx), (iy0, ty), (iz0, tz) = _coords_and_weights(grid_ref[...], D, H, W)

    z_i, y_i = _zy_indices(PB, zc * _ROWS, H)
    mzy = _tap_weight(z_i, iz0, tz) * _tap_weight(y_i, iy0, ty)

    xc = jax.lax.broadcasted_iota(jnp.int32, (PB, W * C), 1)
    wx = _tap_weight(xc // C, ix0, tx)
    k = wx * _replicate_channels(go_ref[...], C, W)

    @pl.when(pb == 0)
    def _():
        gi_ref[...] = jnp.zeros_like(gi_ref)

    gi_ref[...] += jax.lax.dot_general(
        mzy, k, (((0,), (0,)), ((), ())),
        preferred_element_type=jnp.float32)


def _grad_grid_kernel(grid_ref, go_ref, inp_ref, gx_ref, gy_ref, gz_ref,
                      u0_ref, u1_ref, u2_ref, *, D, H, W, C, PB, KC):
    kc = pl.program_id(2)

    (ix0, tx), (iy0, ty), (iz0, tz) = _coords_and_weights(grid_ref[...], D, H, W)

    z_i, y_i = _zy_indices(PB, kc * _ROWS, H)
    wz = _tap_weight(z_i, iz0, tz)
    wy = _tap_weight(y_i, iy0, ty)
    dwz = _tap_dweight(z_i, iz0)
    dwy = _tap_dweight(y_i, iy0)

    a = inp_ref[...]

    @pl.when(kc == 0)
    def _():
        u0_ref[...] = jnp.zeros_like(u0_ref)
        u1_ref[...] = jnp.zeros_like(u1_ref)
        u2_ref[...] = jnp.zeros_like(u2_ref)

    m_all = jnp.concatenate([wz * wy, wz * dwy, dwz * wy], axis=0)
    u_all = jax.lax.dot(m_all, a, preferred_element_type=jnp.float32)
    u0_ref[...] += u_all[:PB, :]
    u1_ref[...] += u_all[PB:2 * PB, :]
    u2_ref[...] += u_all[2 * PB:, :]

    @pl.when(kc == KC - 1)
    def _():
        xc = jax.lax.broadcasted_iota(jnp.int32, (PB, W * C), 1)
        go_rep = _replicate_channels(go_ref[...], C, W)
        k = _tap_weight(xc // C, ix0, tx) * go_rep
        dk = _tap_dweight(xc // C, ix0) * go_rep
        gx_ref[...] = jnp.sum(u0_ref[...] * dk, axis=1, keepdims=True) * (0.5 * (W - 1))
        gy_ref[...] = jnp.sum(u1_ref[...] * k, axis=1, keepdims=True) * (0.5 * (H - 1))
        gz_ref[...] = jnp.sum(u2_ref[...] * k, axis=1, keepdims=True) * (0.5 * (D - 1))


@jax.jit
def _run(grad_output, input, grid):
    N, C, D, H, W = input.shape
    _, Do, Ho, Wo, _ = grid.shape
    P = Do * Ho * Wo

    grid_f = grid.reshape(N, P, 3)
    go_f = jnp.transpose(grad_output.reshape(N, C, P), (0, 2, 1))
    inp_f = jnp.transpose(input, (0, 2, 3, 4, 1)).reshape(N, D * H, W * C)

    n_chunks = (D * H) // _ROWS

    PB1 = 512
    gi_flat = pl.pallas_call(
        functools.partial(_grad_input_kernel, D=D, H=H, W=W, C=C, PB=PB1),
        grid=(N, n_chunks, P // PB1),
        in_specs=[
            pl.BlockSpec((None, PB1, 3), lambda n, zc, p: (n, p, 0)),
            pl.BlockSpec((None, PB1, C), lambda n, zc, p: (n, p, 0)),
        ],
        out_specs=pl.BlockSpec((None, _ROWS, W * C), lambda n, zc, p: (n, zc, 0)),
        out_shape=jax.ShapeDtypeStruct((N, D * H, W * C), jnp.float32),
        compiler_params=pltpu.CompilerParams(
            dimension_semantics=("parallel", "arbitrary", "arbitrary")),
    )(grid_f, go_f)
    grad_input = jnp.transpose(
        gi_flat.reshape(N, D, H, W, C), (0, 4, 1, 2, 3))

    PB2 = 512
    out_sds = jax.ShapeDtypeStruct((N, P, 1), jnp.float32)
    out_spec = pl.BlockSpec((None, PB2, 1), lambda n, p, kc: (n, p, 0))
    u_scratch = pltpu.VMEM((PB2, W * C), jnp.float32)
    gx, gy, gz = pl.pallas_call(
        functools.partial(_grad_grid_kernel, D=D, H=H, W=W, C=C, PB=PB2,
                          KC=n_chunks),
        grid=(N, P // PB2, n_chunks),
        in_specs=[
            pl.BlockSpec((None, PB2, 3), lambda n, p, kc: (n, p, 0)),
            pl.BlockSpec((None, PB2, C), lambda n, p, kc: (n, p, 0)),
            pl.BlockSpec((None, _ROWS, W * C), lambda n, p, kc: (n, kc, 0)),
        ],
        out_specs=(out_spec, out_spec, out_spec),
        out_shape=(out_sds, out_sds, out_sds),
        scratch_shapes=[u_scratch, u_scratch, u_scratch],
        compiler_params=pltpu.CompilerParams(
            dimension_semantics=("parallel", "arbitrary", "arbitrary")),
    )(grid_f, go_f, inp_f)
    grad_grid = jnp.concatenate([gx, gy, gz], axis=-1).reshape(N, Do, Ho, Wo, 3)

    return grad_input, grad_grid


def kernel(grad_output, input, grid, interpolation_mode, padding_mode,
           align_corners, output_mask):
    return _run(grad_output, input, grid)


# speedup vs baseline: 4.3895x; 1.0183x over previous
import functools

import jax
import jax.numpy as jnp
from jax.experimental import pallas as pl
from jax.experimental.pallas import tpu as pltpu

_ROWS = 1024


def _coords_and_weights(grid_blk, D, H, W):
    ix = (grid_blk[:, 0:1] + 1.0) * (0.5 * (W - 1))
    iy = (grid_blk[:, 1:2] + 1.0) * (0.5 * (H - 1))
    iz = (grid_blk[:, 2:3] + 1.0) * (0.5 * (D - 1))
    ix0f = jnp.floor(ix)
    iy0f = jnp.floor(iy)
    iz0f = jnp.floor(iz)
    tx = ix - ix0f
    ty = iy - iy0f
    tz = iz - iz0f
    return (ix0f.astype(jnp.int32), tx), (iy0f.astype(jnp.int32), ty), (iz0f.astype(jnp.int32), tz)


def _tap_weight(idx_lane, i0, t):
    return (jnp.where(idx_lane == i0, 1.0 - t, 0.0)
            + jnp.where(idx_lane == i0 + 1, t, 0.0))


def _tap_dweight(idx_lane, i0):
    return (jnp.where(idx_lane == i0 + 1, 1.0, 0.0)
            - jnp.where(idx_lane == i0, 1.0, 0.0))


def _zy_indices(PB, row0, H):
    zy = jax.lax.broadcasted_iota(jnp.int32, (PB, _ROWS), 1) + row0
    return zy // H, zy % H


def _replicate_channels(go_blk, C, W):
    lane = jax.lax.broadcasted_iota(jnp.int32, (C, W * C), 1)
    sub = jax.lax.broadcasted_iota(jnp.int32, (C, W * C), 0)
    B = ((lane % C) == sub).astype(jnp.float32)
    return jax.lax.dot(go_blk, B, preferred_element_type=jnp.float32)


def _grad_input_kernel(grid_ref, go_ref, gi_ref, *, D, H, W, C, PB):
    zc = pl.program_id(1)
    pb = pl.program_id(2)

    (ix0, tx), (iy0, ty), (iz0, tz) = _coords_and_weights(grid_ref[...], D, H, W)

    z_i, y_i = _zy_indices(PB, zc * _ROWS, H)
    mzy = _tap_weight(z_i, iz0, tz) * _tap_weight(y_i, iy0, ty)

    xc = jax.lax.broadcasted_iota(jnp.int32, (PB, W * C), 1)
    wx = _tap_weight(xc // C, ix0, tx)
    k = wx * _replicate_channels(go_ref[...], C, W)

    @pl.when(pb == 0)
    def _():
        gi_ref[...] = jnp.zeros_like(gi_ref)

    gi_ref[...] += jax.lax.dot_general(
        mzy.astype(jnp.bfloat16), k.astype(jnp.bfloat16),
        (((0,), (0,)), ((), ())),
        preferred_element_type=jnp.float32)


def _grad_grid_kernel(grid_ref, go_ref, inp_ref, gx_ref, gy_ref, gz_ref,
                      u0_ref, u1_ref, u2_ref, *, D, H, W, C, PB, KC):
    kc = pl.program_id(2)

    (ix0, tx), (iy0, ty), (iz0, tz) = _coords_and_weights(grid_ref[...], D, H, W)

    z_i, y_i = _zy_indices(PB, kc * _ROWS, H)
    wz = _tap_weight(z_i, iz0, tz)
    wy = _tap_weight(y_i, iy0, ty)
    dwz = _tap_dweight(z_i, iz0)
    dwy = _tap_dweight(y_i, iy0)

    a = inp_ref[...]

    @pl.when(kc == 0)
    def _():
        u0_ref[...] = jnp.zeros_like(u0_ref)
        u1_ref[...] = jnp.zeros_like(u1_ref)
        u2_ref[...] = jnp.zeros_like(u2_ref)

    m_all = jnp.concatenate([wz * wy, wz * dwy, dwz * wy], axis=0)
    u_all = jax.lax.dot(m_all.astype(jnp.bfloat16), a,
                        preferred_element_type=jnp.float32)
    u0_ref[...] += u_all[:PB, :]
    u1_ref[...] += u_all[PB:2 * PB, :]
    u2_ref[...] += u_all[2 * PB:, :]

    @pl.when(kc == KC - 1)
    def _():
        xc = jax.lax.broadcasted_iota(jnp.int32, (PB, W * C), 1)
        go_rep = _replicate_channels(go_ref[...], C, W)
        k = _tap_weight(xc // C, ix0, tx) * go_rep
        dk = _tap_dweight(xc // C, ix0) * go_rep
        gx_ref[...] = jnp.sum(u0_ref[...] * dk, axis=1, keepdims=True) * (0.5 * (W - 1))
        gy_ref[...] = jnp.sum(u1_ref[...] * k, axis=1, keepdims=True) * (0.5 * (H - 1))
        gz_ref[...] = jnp.sum(u2_ref[...] * k, axis=1, keepdims=True) * (0.5 * (D - 1))


@jax.jit
def _run(grad_output, input, grid):
    N, C, D, H, W = input.shape
    _, Do, Ho, Wo, _ = grid.shape
    P = Do * Ho * Wo

    grid_f = grid.reshape(N, P, 3)
    go_f = jnp.transpose(grad_output.reshape(N, C, P), (0, 2, 1))
    inp_f = jnp.transpose(input, (0, 2, 3, 4, 1)).reshape(
        N, D * H, W * C).astype(jnp.bfloat16)

    n_chunks = (D * H) // _ROWS

    PB1 = 512
    gi_flat = pl.pallas_call(
        functools.partial(_grad_input_kernel, D=D, H=H, W=W, C=C, PB=PB1),
        grid=(N, n_chunks, P // PB1),
        in_specs=[
            pl.BlockSpec((None, PB1, 3), lambda n, zc, p: (n, p, 0)),
            pl.BlockSpec((None, PB1, C), lambda n, zc, p: (n, p, 0)),
        ],
        out_specs=pl.BlockSpec((None, _ROWS, W * C), lambda n, zc, p: (n, zc, 0)),
        out_shape=jax.ShapeDtypeStruct((N, D * H, W * C), jnp.float32),
        compiler_params=pltpu.CompilerParams(
            dimension_semantics=("parallel", "arbitrary", "arbitrary")),
    )(grid_f, go_f)
    grad_input = jnp.transpose(
        gi_flat.reshape(N, D, H, W, C), (0, 4, 1, 2, 3))

    PB2 = 512
    out_sds = jax.ShapeDtypeStruct((N, P, 1), jnp.float32)
    out_spec = pl.BlockSpec((None, PB2, 1), lambda n, p, kc: (n, p, 0))
    u_scratch = pltpu.VMEM((PB2, W * C), jnp.float32)
    gx, gy, gz = pl.pallas_call(
        functools.partial(_grad_grid_kernel, D=D, H=H, W=W, C=C, PB=PB2,
                          KC=n_chunks),
        grid=(N, P // PB2, n_chunks),
        in_specs=[
            pl.BlockSpec((None, PB2, 3), lambda n, p, kc: (n, p, 0)),
            pl.BlockSpec((None, PB2, C), lambda n, p, kc: (n, p, 0)),
            pl.BlockSpec((None, _ROWS, W * C), lambda n, p, kc: (n, kc, 0)),
        ],
        out_specs=(out_spec, out_spec, out_spec),
        out_shape=(out_sds, out_sds, out_sds),
        scratch_shapes=[u_scratch, u_scratch, u_scratch],
        compiler_params=pltpu.CompilerParams(
            dimension_semantics=("parallel", "arbitrary", "arbitrary")),
    )(grid_f, go_f, inp_f)
    grad_grid = jnp.concatenate([gx, gy, gz], axis=-1).reshape(N, Do, Ho, Wo, 3)

    return grad_input, grad_grid


def kernel(grad_output, input, grid, interpolation_mode, padding_mode,
           align_corners, output_mask):
    return _run(grad_output, input, grid)
